# Initial kernel scaffold; baseline (speedup 1.0000x reference)
import jax
import jax.numpy as jnp
from jax.experimental import pallas as pl


def kernel(logits, labels):
    raise NotImplementedError("write your pallas kernel here")


# speedup vs baseline: 21.7237x; 21.7237x over previous
import functools
import math

import jax
import jax.numpy as jnp
from jax import lax
from jax.experimental import pallas as pl
from jax.experimental.pallas import tpu as pltpu
from jax.experimental.pallas import tpu_sc as plsc

_THRESH = -math.log(0.7)
_N_MIN = 262144
_N_TOT = 16 * 512 * 512

_ROWS, _COLS = 32, 128
_NC, _NS, _L = 2, 16, 16
_NW = _NC * _NS
_PER_W = _N_TOT // _NW
_CHUNK = 8192
_NCHUNK = _PER_W // _CHUNK
_VPC = _CHUNK // _L


_RBLK = 32
_NROWCOL = (512, 8192)
_GRID_A = _NROWCOL[0] // _RBLK


def _loss_body(x_ref, y_ref, loss_ref, stats_ref):
    i = pl.program_id(0)
    x = x_ref[...]
    y = y_ref[...]
    loss = jnp.maximum(x, 0.0) - x * y + jnp.log1p(jnp.exp(-jnp.abs(x)))
    loss_ref[...] = loss
    m = loss > _THRESH
    s = jnp.sum(jnp.where(m, loss, 0.0))
    c = jnp.sum(m.astype(jnp.float32))

    @pl.when(i == 0)
    def _init():
        stats_ref[0, 0] = s
        stats_ref[0, 1] = c

    @pl.when(i > 0)
    def _acc():
        stats_ref[0, 0] += s
        stats_ref[0, 1] += c

    @pl.when(i == _GRID_A - 1)
    def _fin():
        stats_ref[0, 2] = stats_ref[0, 0] / jnp.maximum(stats_ref[0, 1], 1.0)


def _tc_loss(logits, labels):
    x = logits.reshape(_NROWCOL)
    y = labels.reshape(_NROWCOL)
    loss, stats = pl.pallas_call(
        _loss_body,
        grid=(_GRID_A,),
        in_specs=[
            pl.BlockSpec((_RBLK, _NROWCOL[1]), lambda i: (i, 0)),
            pl.BlockSpec((_RBLK, _NROWCOL[1]), lambda i: (i, 0)),
        ],
        out_specs=[
            pl.BlockSpec((_RBLK, _NROWCOL[1]), lambda i: (i, 0)),
            pl.BlockSpec(memory_space=pltpu.SMEM),
        ],
        out_shape=[
            jax.ShapeDtypeStruct(_NROWCOL, jnp.float32),
            jax.ShapeDtypeStruct((1, 16), jnp.float32),
        ],
    )(x, y)
    return loss.reshape(-1), stats


_MESH = plsc.VectorSubcoreMesh(core_axis_name="c", subcore_axis_name="s",
                               num_cores=_NC, num_subcores=_NS)


def _zero_hist(cnt_ref, sum_ref):
    z_i = jnp.zeros((_L,), jnp.int32)
    z_f = jnp.zeros((_L,), jnp.float32)

    def body(k, carry):
        r = k >> 3
        c = (k & 7) * _L
        cnt_ref[r, pl.ds(c, _L)] = z_i
        sum_ref[r, pl.ds(c, _L)] = z_f
        return carry

    lax.fori_loop(0, _ROWS * 8, body, 0)


def _fill_rowidx(rowidx_ref):
    iota = lax.iota(jnp.int32, _L)
    rowidx_ref[pl.ds(0, _L)] = iota
    rowidx_ref[pl.ds(_L, _L)] = iota + _L


def _merge_pair(dst_c, dst_s, tmp_c, tmp_s):
    def body(k, carry):
        r = k >> 3
        c = (k & 7) * _L
        dst_c[r, pl.ds(c, _L)] += tmp_c[r, pl.ds(c, _L)]
        dst_s[r, pl.ds(c, _L)] += tmp_s[r, pl.ds(c, _L)]
        return carry

    lax.fori_loop(0, _ROWS * 8, body, 0)


def _sval(v, i):
    return jnp.squeeze(lax.slice(v, (i,), (i + 1,)))


def _scan_hist(cnt_ref, sum_ref, q):
    lanes = lax.iota(jnp.int32, _L)

    def body(k, carry):
        found, b, pcb, psb, cum_c, cum_s = carry
        r = k >> 3
        c = (k & 7) * _L
        vc = cnt_ref[r, pl.ds(c, _L)]
        vs = sum_ref[r, pl.ds(c, _L)]
        cs_c = plsc.cumsum(vc) + cum_c
        cs_s = plsc.cumsum(vs) + cum_s
        crossed = cs_c > q
        has = _sval(plsc.all_reduce_population_count(crossed), 0) > 0
        lane = _sval(plsc.all_reduce_ffs(crossed), 0)
        sel = lanes == lane
        pcx = jnp.sum(jnp.where(sel, cs_c, 0))
        psx = jnp.sum(jnp.where(sel, cs_s, 0.0))
        take = jnp.logical_and(has, found == 0)
        b = jnp.where(take, k * _L + lane, b)
        pcb = jnp.where(take, pcx, pcb)
        psb = jnp.where(take, psx, psb)
        found = jnp.where(take, 1, found)
        return (found, b, pcb, psb,
                cum_c + jnp.sum(vc), cum_s + jnp.sum(vs))

    init = (jnp.int32(0), jnp.int32(0), jnp.int32(0), jnp.float32(0.0),
            jnp.int32(0), jnp.float32(0.0))
    found, b, pcb, psb, tot_c, tot_s = lax.fori_loop(0, _ROWS * 8, body, init)
    return b, pcb, psb, tot_c, tot_s


def _stream_chunks(loss_hbm, buf, sems, base, process_vec):
    copies = [None, None]
    copies[0] = pltpu.async_copy(
        loss_hbm.at[pl.ds(base, _CHUNK)], buf.at[0], sems[0])
    for k in range(_NCHUNK):
        cur = k % 2
        if k + 1 < _NCHUNK:
            nxt = (k + 1) % 2
            copies[nxt] = pltpu.async_copy(
                loss_hbm.at[pl.ds(base + (k + 1) * _CHUNK, _CHUNK)],
                buf.at[nxt], sems[nxt])
        copies[cur].wait()

        def body(i, carry):
            process_vec(buf[cur, pl.ds(i * _L, _L)])
            return carry

        lax.fori_loop(0, _VPC, body, 0)


@functools.partial(
    pl.kernel,
    out_type=[
        jax.ShapeDtypeStruct((_NC, _ROWS, _COLS), jnp.int32),
        jax.ShapeDtypeStruct((_NC, _ROWS, _COLS), jnp.float32),
    ],
    mesh=_MESH,
    compiler_params=pltpu.CompilerParams(needs_layout_passes=False),
    scratch_types=[
        pltpu.VMEM((2, _CHUNK), jnp.float32),
        pltpu.VMEM((_ROWS, _COLS), jnp.int32),
        pltpu.VMEM((_ROWS, _COLS), jnp.float32),
        pltpu.VMEM((_ROWS,), jnp.int32),
        pltpu.VMEM_SHARED((_ROWS, _COLS), jnp.int32),
        pltpu.VMEM_SHARED((_ROWS, _COLS), jnp.float32),
        pltpu.SemaphoreType.DMA,
        pltpu.SemaphoreType.DMA,
    ],
)
def _sc_hist1(loss_hbm, outc_hbm, outs_hbm,
              buf, h1c, h1s, rowidx, shc, shs, sem0, sem1):
    cid = lax.axis_index("c")
    sid = lax.axis_index("s")
    _zero_hist(h1c, h1s)
    _fill_rowidx(rowidx)

    @pl.when(sid == 0)
    def _zero_shared():
        pltpu.sync_copy(h1c, shc)
        pltpu.sync_copy(h1s, shs)

    plsc.subcore_barrier()

    ones = jnp.ones((_L,), jnp.int32)
    base = (sid * _NC + cid) * _PER_W

    def process(v):
        p = lax.bitcast_convert_type(v, jnp.int32)
        b = jnp.right_shift(p, 19)
        r = jnp.right_shift(b, 7)
        c = jnp.bitwise_and(b, 127)
        plsc.addupdate_scatter(h1c, [r, c], ones)
        plsc.addupdate_scatter(h1s, [r, c], v)

    _stream_chunks(loss_hbm, buf, (sem0, sem1), base, process)

    pltpu.sync_copy(h1c, shc.at[rowidx], add=True)
    pltpu.sync_copy(h1s, shs.at[rowidx], add=True)
    plsc.subcore_barrier()

    @pl.when(sid == 0)
    def _write_out():
        pltpu.sync_copy(shc, outc_hbm.at[cid])
        pltpu.sync_copy(shs, outs_hbm.at[cid])


@functools.partial(
    pl.kernel,
    out_type=[
        jax.ShapeDtypeStruct((_NC, _ROWS, _COLS), jnp.int32),
        jax.ShapeDtypeStruct((_NC, _ROWS, _COLS), jnp.float32),
        jax.ShapeDtypeStruct((_L,), jnp.int32),
        jax.ShapeDtypeStruct((_L,), jnp.float32),
    ],
    mesh=_MESH,
    compiler_params=pltpu.CompilerParams(needs_layout_passes=False),
    scratch_types=[
        pltpu.VMEM((2, _CHUNK), jnp.float32),
        pltpu.VMEM((_ROWS, _COLS), jnp.int32),
        pltpu.VMEM((_ROWS, _COLS), jnp.float32),
        pltpu.VMEM((_ROWS, _COLS), jnp.int32),
        pltpu.VMEM((_ROWS, _COLS), jnp.float32),
        pltpu.VMEM((_ROWS,), jnp.int32),
        pltpu.VMEM((_L,), jnp.int32),
        pltpu.VMEM((_L,), jnp.float32),
        pltpu.VMEM_SHARED((_ROWS, _COLS), jnp.int32),
        pltpu.VMEM_SHARED((_ROWS, _COLS), jnp.float32),
        pltpu.SemaphoreType.DMA,
        pltpu.SemaphoreType.DMA,
    ],
)
def _sc_hist2(loss_hbm, h1c_hbm, h1s_hbm,
              outc_hbm, outs_hbm, auxi_hbm, auxf_hbm,
              buf, h2c, h2s, tmpc, tmps, rowidx, auxi_v, auxf_v,
              shc, shs, sem0, sem1):
    cid = lax.axis_index("c")
    sid = lax.axis_index("s")
    _fill_rowidx(rowidx)

    pltpu.sync_copy(h1c_hbm.at[0], h2c)
    pltpu.sync_copy(h1s_hbm.at[0], h2s)
    pltpu.sync_copy(h1c_hbm.at[1], tmpc)
    pltpu.sync_copy(h1s_hbm.at[1], tmps)
    _merge_pair(h2c, h2s, tmpc, tmps)

    b1, pc1, ps1, tot_c, tot_s = _scan_hist(h2c, h2s, jnp.int32(_N_TOT - _N_MIN))
    c_above = tot_c - pc1
    s_above = tot_s - ps1
    j_need = _N_MIN - c_above

    @pl.when(jnp.logical_and(sid == 0, cid == 0))
    def _write_aux():
        zi = jnp.zeros((_L,), jnp.int32)
        auxi_v[pl.ds(0, _L)] = zi + jnp.where(lax.iota(jnp.int32, _L) == 0,
                                              b1, j_need)
        auxf_v[pl.ds(0, _L)] = jnp.zeros((_L,), jnp.float32) + s_above
        pltpu.sync_copy(auxi_v, auxi_hbm)
        pltpu.sync_copy(auxf_v, auxf_hbm)

    _zero_hist(h2c, h2s)

    @pl.when(sid == 0)
    def _zero_shared():
        pltpu.sync_copy(h2c, shc)
        pltpu.sync_copy(h2s, shs)

    plsc.subcore_barrier()

    ones = jnp.ones((_L,), jnp.int32)
    base = (sid * _NC + cid) * _PER_W

    def process(v):
        p = lax.bitcast_convert_type(v, jnp.int32)
        b = jnp.right_shift(p, 19)
        meq = b == b1
        sub = jnp.bitwise_and(jnp.right_shift(p, 7), 4095)
        r = jnp.right_shift(sub, 7)
        c = jnp.bitwise_and(sub, 127)
        plsc.addupdate_scatter(h2c, [r, c], ones, mask=meq)
        plsc.addupdate_scatter(h2s, [r, c], v, mask=meq)

    _stream_chunks(loss_hbm, buf, (sem0, sem1), base, process)

    pltpu.sync_copy(h2c, shc.at[rowidx], add=True)
    pltpu.sync_copy(h2s, shs.at[rowidx], add=True)
    plsc.subcore_barrier()

    @pl.when(sid == 0)
    def _write_out():
        pltpu.sync_copy(shc, outc_hbm.at[cid])
        pltpu.sync_copy(shs, outs_hbm.at[cid])


@functools.partial(
    pl.kernel,
    out_type=jax.ShapeDtypeStruct((_L,), jnp.float32),
    mesh=_MESH,
    compiler_params=pltpu.CompilerParams(needs_layout_passes=False),
    scratch_types=[
        pltpu.VMEM((_ROWS, _COLS), jnp.int32),
        pltpu.VMEM((_ROWS, _COLS), jnp.float32),
        pltpu.VMEM((_ROWS, _COLS), jnp.int32),
        pltpu.VMEM((_ROWS, _COLS), jnp.float32),
        pltpu.VMEM((1, _L), jnp.float32),
        pltpu.VMEM((_L,), jnp.int32),
        pltpu.VMEM((_L,), jnp.float32),
        pltpu.VMEM((_L,), jnp.float32),
    ],
)
def _sc_final(h2c_hbm, h2s_hbm, auxi_hbm, auxf_hbm, stats_hbm, out_hbm,
              mc, ms, tmpc, tmps, stats_v, auxi_v, auxf_v, out_v):
    cid = lax.axis_index("c")
    sid = lax.axis_index("s")

    @pl.when(jnp.logical_and(sid == 0, cid == 0))
    def _go():
        pltpu.sync_copy(h2c_hbm.at[0], mc)
        pltpu.sync_copy(h2s_hbm.at[0], ms)
        pltpu.sync_copy(h2c_hbm.at[1], tmpc)
        pltpu.sync_copy(h2s_hbm.at[1], tmps)
        _merge_pair(mc, ms, tmpc, tmps)
        pltpu.sync_copy(auxi_hbm, auxi_v)
        pltpu.sync_copy(auxf_hbm, auxf_v)
        pltpu.sync_copy(stats_hbm, stats_v)

        auxi = auxi_v[pl.ds(0, _L)]
        b1 = _sval(auxi, 0)
        j_need = _sval(auxi, 1)
        s_above = _sval(auxf_v[pl.ds(0, _L)], 0)

        _, _, _, tot_c2, tot_s2 = _scan_hist(mc, ms, jnp.int32(0x7FFFFFF0))
        c2, pc2, ps2, _, _ = _scan_hist(mc, ms, tot_c2 - j_need)
        a2 = tot_c2 - pc2
        s2 = tot_s2 - ps2
        rem = j_need - a2

        rep_bits = (jnp.zeros((_L,), jnp.int32)
                    + (jnp.left_shift(b1, 19) | jnp.left_shift(c2, 7)))
        rep = _sval(lax.bitcast_convert_type(rep_bits, jnp.float32), 0)
        sum_topk = s_above + s2 + rem.astype(jnp.float32) * rep
        mean_topk = sum_topk * jnp.float32(1.0 / _N_MIN)

        stats = stats_v[0, pl.ds(0, _L)]
        cnt_gt = _sval(stats, 1)
        mean_gt = _sval(stats, 2)

        final = jnp.where(cnt_gt > jnp.float32(_N_MIN), mean_gt, mean_topk)
        out_v[pl.ds(0, _L)] = jnp.zeros((_L,), jnp.float32) + final
        pltpu.sync_copy(out_v, out_hbm)


def kernel(logits, labels):
    loss, stats = _tc_loss(logits, labels)
    h1c, h1s = _sc_hist1(loss)
    h2c, h2s, auxi, auxf = _sc_hist2(loss, h1c, h1s)
    out = _sc_final(h2c, h2s, auxi, auxf, stats)
    return out[0]


# speedup vs baseline: 49.2558x; 2.2674x over previous
import functools
import math

import jax
import jax.numpy as jnp
from jax import lax
from jax.experimental import pallas as pl
from jax.experimental.pallas import tpu as pltpu
from jax.experimental.pallas import tpu_sc as plsc

_THRESH = -math.log(0.7)
_N_MIN = 262144
_N_TOT = 16 * 512 * 512

_ROWS, _COLS = 32, 128
_NC, _NS, _L = 2, 16, 16
_NW = _NC * _NS
_PER_W = _N_TOT // _NW
_CHUNK = 8192
_NCHUNK = _PER_W // _CHUNK
_VPC = _CHUNK // _L


_NROWCOL = (8192, 512)
_RBLK = 512
_GRID_A = _NROWCOL[0] // _RBLK


def _loss_body(x_ref, y_ref, loss_ref, stats_ref):
    i = pl.program_id(0)
    x = x_ref[...]
    y = y_ref[...]
    loss = jnp.maximum(x, 0.0) - x * y + jnp.log1p(jnp.exp(-jnp.abs(x)))
    loss_ref[...] = loss
    m = loss > _THRESH
    s = jnp.sum(jnp.where(m, loss, 0.0))
    c = jnp.sum(m.astype(jnp.float32))

    @pl.when(i == 0)
    def _init():
        stats_ref[0, 0] = s
        stats_ref[0, 1] = c

    @pl.when(i > 0)
    def _acc():
        stats_ref[0, 0] += s
        stats_ref[0, 1] += c

    @pl.when(i == _GRID_A - 1)
    def _fin():
        stats_ref[0, 2] = stats_ref[0, 0] / jnp.maximum(stats_ref[0, 1], 1.0)


def _tc_loss(logits, labels):
    x = logits.reshape(_NROWCOL)
    y = labels.reshape(_NROWCOL)
    loss, stats = pl.pallas_call(
        _loss_body,
        grid=(_GRID_A,),
        in_specs=[
            pl.BlockSpec((_RBLK, _NROWCOL[1]), lambda i: (i, 0)),
            pl.BlockSpec((_RBLK, _NROWCOL[1]), lambda i: (i, 0)),
        ],
        out_specs=[
            pl.BlockSpec((_RBLK, _NROWCOL[1]), lambda i: (i, 0)),
            pl.BlockSpec(memory_space=pltpu.SMEM),
        ],
        out_shape=[
            jax.ShapeDtypeStruct(_NROWCOL, jnp.float32),
            jax.ShapeDtypeStruct((1, 16), jnp.float32),
        ],
    )(x, y)
    return loss.reshape(-1), stats


_MESH = plsc.VectorSubcoreMesh(core_axis_name="c", subcore_axis_name="s",
                               num_cores=_NC, num_subcores=_NS)
_SC_PARAMS = pltpu.CompilerParams(needs_layout_passes=False)


def _zero_hist2(cnt_ref, sum_ref):
    z_i = jnp.zeros((_L,), jnp.int32)
    z_f = jnp.zeros((_L,), jnp.float32)

    @plsc.parallel_loop(0, _ROWS * 8, unroll=4)
    def _(k):
        r = k >> 3
        c = (k & 7) * _L
        cnt_ref[r, pl.ds(c, _L)] = z_i
        sum_ref[r, pl.ds(c, _L)] = z_f


def _zero_hist1(cnt_ref):
    z_i = jnp.zeros((_L,), jnp.int32)

    @plsc.parallel_loop(0, _ROWS * 8, unroll=4)
    def _(k):
        r = k >> 3
        c = (k & 7) * _L
        cnt_ref[r, pl.ds(c, _L)] = z_i


def _fill_rowidx(rowidx_ref):
    iota = lax.iota(jnp.int32, _L)
    rowidx_ref[pl.ds(0, _L)] = iota
    rowidx_ref[pl.ds(_L, _L)] = iota + _L


def _merge_into(dst_refs, src_refs):
    @plsc.parallel_loop(0, _ROWS * 8, unroll=4)
    def _(k):
        r = k >> 3
        c = (k & 7) * _L
        for d, s in zip(dst_refs, src_refs):
            d[r, pl.ds(c, _L)] += s[r, pl.ds(c, _L)]


def _sval(v, i):
    return jnp.squeeze(lax.slice(v, (i,), (i + 1,)))


def _scan_hist(cnt_ref, sum_ref, q):
    lanes = lax.iota(jnp.int32, _L)
    use_sum = sum_ref is not None

    def body(k, carry):
        found, b, pcb, psb, cum_c, cum_s = carry
        r = k >> 3
        c = (k & 7) * _L
        vc = cnt_ref[r, pl.ds(c, _L)]
        cs_c = plsc.cumsum(vc) + cum_c
        crossed = cs_c > q
        has = _sval(plsc.all_reduce_population_count(crossed), 0) > 0
        lane = _sval(plsc.all_reduce_ffs(crossed), 0)
        sel = lanes == lane
        pcx = jnp.sum(jnp.where(sel, cs_c, 0))
        take = jnp.logical_and(has, found == 0)
        b = jnp.where(take, k * _L + lane, b)
        pcb = jnp.where(take, pcx, pcb)
        found = jnp.where(take, 1, found)
        cum_c = cum_c + jnp.sum(vc)
        if use_sum:
            vs = sum_ref[r, pl.ds(c, _L)]
            cs_s = plsc.cumsum(vs) + cum_s
            psx = jnp.sum(jnp.where(sel, cs_s, 0.0))
            psb = jnp.where(take, psx, psb)
            cum_s = cum_s + jnp.sum(vs)
        return (found, b, pcb, psb, cum_c, cum_s)

    init = (jnp.int32(0), jnp.int32(0), jnp.int32(0), jnp.float32(0.0),
            jnp.int32(0), jnp.float32(0.0))
    found, b, pcb, psb, tot_c, tot_s = lax.fori_loop(0, _ROWS * 8, body, init)
    return b, pcb, psb, tot_c, tot_s


def _stream_chunks(loss_hbm, buf, sems, base, process_vec, carry_init):
    copies = [None, None]
    copies[0] = pltpu.async_copy(
        loss_hbm.at[pl.ds(base, _CHUNK)], buf.at[0], sems[0])
    carry = carry_init
    for k in range(_NCHUNK):
        cur = k % 2
        if k + 1 < _NCHUNK:
            nxt = (k + 1) % 2
            copies[nxt] = pltpu.async_copy(
                loss_hbm.at[pl.ds(base + (k + 1) * _CHUNK, _CHUNK)],
                buf.at[nxt], sems[nxt])
        copies[cur].wait()

        @plsc.parallel_loop(0, _VPC, unroll=8, carry=carry)
        def carry(i, cy):
            return process_vec(buf[cur, pl.ds(i * _L, _L)], cy)

    return carry


@functools.partial(
    pl.kernel,
    out_type=jax.ShapeDtypeStruct((_NC, _ROWS, _COLS), jnp.int32),
    mesh=_MESH,
    compiler_params=_SC_PARAMS,
    scratch_types=[
        pltpu.VMEM((2, _CHUNK), jnp.float32),
        pltpu.VMEM((_ROWS, _COLS), jnp.int32),
        pltpu.VMEM((_ROWS,), jnp.int32),
        pltpu.VMEM_SHARED((_ROWS, _COLS), jnp.int32),
        pltpu.SemaphoreType.DMA,
        pltpu.SemaphoreType.DMA,
    ],
)
def _sc_hist1(loss_hbm, outc_hbm, buf, h1c, rowidx, shc, sem0, sem1):
    cid = lax.axis_index("c")
    sid = lax.axis_index("s")
    _zero_hist1(h1c)
    _fill_rowidx(rowidx)

    @pl.when(sid == 0)
    def _zero_shared():
        pltpu.sync_copy(h1c, shc)

    plsc.subcore_barrier()

    ones = jnp.ones((_L,), jnp.int32)
    base = (sid * _NC + cid) * _PER_W

    def process(v, cy):
        p = lax.bitcast_convert_type(v, jnp.int32)
        b = jnp.right_shift(p, 19)
        r = jnp.right_shift(b, 7)
        c = jnp.bitwise_and(b, 127)
        plsc.addupdate_scatter(h1c, [r, c], ones)
        return cy

    _stream_chunks(loss_hbm, buf, (sem0, sem1), base, process,
                   jnp.int32(0))

    pltpu.sync_copy(h1c, shc.at[rowidx], add=True)
    plsc.subcore_barrier()

    @pl.when(sid == 0)
    def _write_out():
        pltpu.sync_copy(shc, outc_hbm.at[cid])


@functools.partial(
    pl.kernel,
    out_type=[
        jax.ShapeDtypeStruct((_NC, _ROWS, _COLS), jnp.int32),
        jax.ShapeDtypeStruct((_NC, _ROWS, _COLS), jnp.float32),
        jax.ShapeDtypeStruct((_L,), jnp.int32),
        jax.ShapeDtypeStruct((_NC, _L), jnp.float32),
    ],
    mesh=_MESH,
    compiler_params=_SC_PARAMS,
    scratch_types=[
        pltpu.VMEM((2, _CHUNK), jnp.float32),
        pltpu.VMEM((_ROWS, _COLS), jnp.int32),
        pltpu.VMEM((_ROWS, _COLS), jnp.float32),
        pltpu.VMEM((_ROWS, _COLS), jnp.int32),
        pltpu.VMEM((_ROWS,), jnp.int32),
        pltpu.VMEM((_L,), jnp.int32),
        pltpu.VMEM((_L,), jnp.float32),
        pltpu.VMEM((_NS, _L), jnp.float32),
        pltpu.VMEM_SHARED((_ROWS, _COLS), jnp.int32),
        pltpu.VMEM_SHARED((_ROWS, _COLS), jnp.float32),
        pltpu.VMEM_SHARED((_NS, _L), jnp.float32),
        pltpu.SemaphoreType.DMA,
        pltpu.SemaphoreType.DMA,
    ],
)
def _sc_hist2(loss_hbm, h1c_hbm,
              outc_hbm, outs_hbm, auxi_hbm, auxf_hbm,
              buf, h2c, h2s, tmpc, rowidx, auxi_v, auxf_v, acc16,
              shc, shs, sh_acc, sem0, sem1):
    cid = lax.axis_index("c")
    sid = lax.axis_index("s")
    _fill_rowidx(rowidx)

    pltpu.sync_copy(h1c_hbm.at[0], h2c)
    pltpu.sync_copy(h1c_hbm.at[1], tmpc)
    _merge_into([h2c], [tmpc])

    b1, pc1, _, tot_c, _ = _scan_hist(h2c, None, jnp.int32(_N_TOT - _N_MIN))
    c_above = tot_c - pc1
    j_need = _N_MIN - c_above

    @pl.when(jnp.logical_and(sid == 0, cid == 0))
    def _write_aux():
        auxi_v[pl.ds(0, _L)] = (jnp.zeros((_L,), jnp.int32)
                                + jnp.where(lax.iota(jnp.int32, _L) == 0,
                                            b1, j_need))
        pltpu.sync_copy(auxi_v, auxi_hbm)

    _zero_hist2(h2c, h2s)

    @pl.when(sid == 0)
    def _zero_shared():
        pltpu.sync_copy(h2c, shc)
        pltpu.sync_copy(h2s, shs)

    plsc.subcore_barrier()

    ones = jnp.ones((_L,), jnp.int32)
    zf = jnp.zeros((_L,), jnp.float32)
    base = (sid * _NC + cid) * _PER_W

    def process(v, acc):
        p = lax.bitcast_convert_type(v, jnp.int32)
        b = jnp.right_shift(p, 19)
        meq = b == b1
        acc = acc + jnp.where(b > b1, v, zf)
        sub = jnp.bitwise_and(jnp.right_shift(p, 7), 4095)
        r = jnp.right_shift(sub, 7)
        c = jnp.bitwise_and(sub, 127)
        plsc.addupdate_scatter(h2c, [r, c], ones, mask=meq)
        plsc.addupdate_scatter(h2s, [r, c], v, mask=meq)
        return acc

    acc = _stream_chunks(loss_hbm, buf, (sem0, sem1), base, process, zf)

    auxf_v[pl.ds(0, _L)] = acc
    pltpu.sync_copy(auxf_v, sh_acc.at[sid])
    pltpu.sync_copy(h2c, shc.at[rowidx], add=True)
    pltpu.sync_copy(h2s, shs.at[rowidx], add=True)
    plsc.subcore_barrier()

    @pl.when(sid == 0)
    def _write_out():
        pltpu.sync_copy(shc, outc_hbm.at[cid])
        pltpu.sync_copy(shs, outs_hbm.at[cid])
        pltpu.sync_copy(sh_acc, acc16)
        total = acc16[0, pl.ds(0, _L)]
        for s in range(1, _NS):
            total = total + acc16[s, pl.ds(0, _L)]
        auxf_v[pl.ds(0, _L)] = total
        pltpu.sync_copy(auxf_v, auxf_hbm.at[cid])


@functools.partial(
    pl.kernel,
    out_type=jax.ShapeDtypeStruct((_L,), jnp.float32),
    mesh=_MESH,
    compiler_params=_SC_PARAMS,
    scratch_types=[
        pltpu.VMEM((_ROWS, _COLS), jnp.int32),
        pltpu.VMEM((_ROWS, _COLS), jnp.float32),
        pltpu.VMEM((_ROWS, _COLS), jnp.int32),
        pltpu.VMEM((_ROWS, _COLS), jnp.float32),
        pltpu.VMEM((1, _L), jnp.float32),
        pltpu.VMEM((_L,), jnp.int32),
        pltpu.VMEM((_NC, _L), jnp.float32),
        pltpu.VMEM((_L,), jnp.float32),
    ],
)
def _sc_final(h2c_hbm, h2s_hbm, auxi_hbm, auxf_hbm, stats_hbm, out_hbm,
              mc, ms, tmpc, tmps, stats_v, auxi_v, auxf_v, out_v):
    cid = lax.axis_index("c")
    sid = lax.axis_index("s")

    @pl.when(jnp.logical_and(sid == 0, cid == 0))
    def _go():
        pltpu.sync_copy(h2c_hbm.at[0], mc)
        pltpu.sync_copy(h2s_hbm.at[0], ms)
        pltpu.sync_copy(h2c_hbm.at[1], tmpc)
        pltpu.sync_copy(h2s_hbm.at[1], tmps)
        _merge_into([mc, ms], [tmpc, tmps])
        pltpu.sync_copy(auxi_hbm, auxi_v)
        pltpu.sync_copy(auxf_hbm, auxf_v)
        pltpu.sync_copy(stats_hbm, stats_v)

        auxi = auxi_v[pl.ds(0, _L)]
        b1 = _sval(auxi, 0)
        j_need = _sval(auxi, 1)
        s_above_v = auxf_v[0, pl.ds(0, _L)] + auxf_v[1, pl.ds(0, _L)]
        s_above = jnp.sum(s_above_v)

        _, _, _, tot_c2, tot_s2 = _scan_hist(mc, ms, jnp.int32(0x7FFFFFF0))
        c2, pc2, ps2, _, _ = _scan_hist(mc, ms, tot_c2 - j_need)
        a2 = tot_c2 - pc2
        s2 = tot_s2 - ps2
        rem = j_need - a2

        rep_bits = (jnp.zeros((_L,), jnp.int32)
                    + (jnp.left_shift(b1, 19) | jnp.left_shift(c2, 7)))
        rep = _sval(lax.bitcast_convert_type(rep_bits, jnp.float32), 0)
        sum_topk = s_above + s2 + rem.astype(jnp.float32) * rep
        mean_topk = sum_topk * jnp.float32(1.0 / _N_MIN)

        stats = stats_v[0, pl.ds(0, _L)]
        cnt_gt = _sval(stats, 1)
        mean_gt = _sval(stats, 2)

        final = jnp.where(cnt_gt > jnp.float32(_N_MIN), mean_gt, mean_topk)
        out_v[pl.ds(0, _L)] = jnp.zeros((_L,), jnp.float32) + final
        pltpu.sync_copy(out_v, out_hbm)


def kernel(logits, labels):
    loss, stats = _tc_loss(logits, labels)
    h1c = _sc_hist1(loss)
    h2c, h2s, auxi, auxf = _sc_hist2(loss, h1c)
    out = _sc_final(h2c, h2s, auxi, auxf, stats)
    return out[0]


# speedup vs baseline: 59.9676x; 1.2175x over previous
import functools
import math

import jax
import jax.numpy as jnp
from jax import lax
from jax.experimental import pallas as pl
from jax.experimental.pallas import tpu as pltpu
from jax.experimental.pallas import tpu_sc as plsc

_THRESH = -math.log(0.7)
_N_MIN = 262144
_N_TOT = 16 * 512 * 512

_ROWS, _COLS = 32, 128
_NC, _NS, _L = 2, 16, 16
_NW = _NC * _NS
_PER_W = _N_TOT // _NW
_CHUNK = 8192
_NCHUNK = _PER_W // _CHUNK
_VPC = _CHUNK // _L
_LCOL = 512
_CROWS = _CHUNK // _LCOL
_WROWS = _PER_W // _LCOL


_NROWCOL = (8192, 512)
_RBLK = 512
_GRID_A = _NROWCOL[0] // _RBLK


def _loss_body(x_ref, y_ref, loss_ref, stats_ref):
    i = pl.program_id(0)
    x = x_ref[...]
    y = y_ref[...]
    loss = jnp.maximum(x, 0.0) - x * y + jnp.log1p(jnp.exp(-jnp.abs(x)))
    loss_ref[...] = loss
    m = loss > _THRESH
    s = jnp.sum(jnp.where(m, loss, 0.0))
    c = jnp.sum(m.astype(jnp.float32))

    @pl.when(i == 0)
    def _init():
        stats_ref[0, 0] = s
        stats_ref[0, 1] = c

    @pl.when(i > 0)
    def _acc():
        stats_ref[0, 0] += s
        stats_ref[0, 1] += c

    @pl.when(i == _GRID_A - 1)
    def _fin():
        stats_ref[0, 2] = stats_ref[0, 0] / jnp.maximum(stats_ref[0, 1], 1.0)


def _tc_loss(logits, labels):
    x = logits.reshape(_NROWCOL)
    y = labels.reshape(_NROWCOL)
    loss, stats = pl.pallas_call(
        _loss_body,
        grid=(_GRID_A,),
        in_specs=[
            pl.BlockSpec((_RBLK, _NROWCOL[1]), lambda i: (i, 0)),
            pl.BlockSpec((_RBLK, _NROWCOL[1]), lambda i: (i, 0)),
        ],
        out_specs=[
            pl.BlockSpec((_RBLK, _NROWCOL[1]), lambda i: (i, 0)),
            pl.BlockSpec(memory_space=pltpu.SMEM),
        ],
        out_shape=[
            jax.ShapeDtypeStruct(_NROWCOL, jnp.float32),
            jax.ShapeDtypeStruct((1, 16), jnp.float32),
        ],
    )(x, y)
    return loss, stats


_MESH = plsc.VectorSubcoreMesh(core_axis_name="c", subcore_axis_name="s",
                               num_cores=_NC, num_subcores=_NS)
_SC_PARAMS = pltpu.CompilerParams(needs_layout_passes=False)


def _zero_hist2(cnt_ref, sum_ref):
    z_i = jnp.zeros((_L,), jnp.int32)
    z_f = jnp.zeros((_L,), jnp.float32)

    @plsc.parallel_loop(0, _ROWS * 8, unroll=4)
    def _(k):
        r = k >> 3
        c = (k & 7) * _L
        cnt_ref[r, pl.ds(c, _L)] = z_i
        sum_ref[r, pl.ds(c, _L)] = z_f


def _zero_hist1(cnt_ref):
    z_i = jnp.zeros((_L,), jnp.int32)

    @plsc.parallel_loop(0, _ROWS * 8, unroll=4)
    def _(k):
        r = k >> 3
        c = (k & 7) * _L
        cnt_ref[r, pl.ds(c, _L)] = z_i


def _fill_rowidx(rowidx_ref):
    iota = lax.iota(jnp.int32, _L)
    rowidx_ref[pl.ds(0, _L)] = iota
    rowidx_ref[pl.ds(_L, _L)] = iota + _L


def _merge_into(dst_refs, src_refs):
    @plsc.parallel_loop(0, _ROWS * 8, unroll=4)
    def _(k):
        r = k >> 3
        c = (k & 7) * _L
        for d, s in zip(dst_refs, src_refs):
            d[r, pl.ds(c, _L)] += s[r, pl.ds(c, _L)]


def _sval(v, i):
    return jnp.squeeze(lax.slice(v, (i,), (i + 1,)))


def _scan_hist(cnt_ref, sum_ref, q):
    lanes = lax.iota(jnp.int32, _L)
    use_sum = sum_ref is not None

    def body(k, carry):
        found, b, pcb, psb, cum_c, cum_s = carry
        r = k >> 3
        c = (k & 7) * _L
        vc = cnt_ref[r, pl.ds(c, _L)]
        cs_c = plsc.cumsum(vc) + cum_c
        crossed = cs_c > q
        has = _sval(plsc.all_reduce_population_count(crossed), 0) > 0
        lane = _sval(plsc.all_reduce_ffs(crossed), 0)
        sel = lanes == lane
        pcx = jnp.sum(jnp.where(sel, cs_c, 0))
        take = jnp.logical_and(has, found == 0)
        b = jnp.where(take, k * _L + lane, b)
        pcb = jnp.where(take, pcx, pcb)
        found = jnp.where(take, 1, found)
        cum_c = cum_c + jnp.sum(vc)
        if use_sum:
            vs = sum_ref[r, pl.ds(c, _L)]
            cs_s = plsc.cumsum(vs) + cum_s
            psx = jnp.sum(jnp.where(sel, cs_s, 0.0))
            psb = jnp.where(take, psx, psb)
            cum_s = cum_s + jnp.sum(vs)
        return (found, b, pcb, psb, cum_c, cum_s)

    init = (jnp.int32(0), jnp.int32(0), jnp.int32(0), jnp.float32(0.0),
            jnp.int32(0), jnp.float32(0.0))
    found, b, pcb, psb, tot_c, tot_s = lax.fori_loop(0, _ROWS * 8, body, init)
    return b, pcb, psb, tot_c, tot_s


def _stream_chunks(loss_hbm, buf, sems, base, process_vec, carry_init):
    copies = [None, None]
    copies[0] = pltpu.async_copy(
        loss_hbm.at[pl.ds(base, _CROWS)], buf.at[0], sems[0])
    carry = carry_init
    for k in range(_NCHUNK):
        cur = k % 2
        if k + 1 < _NCHUNK:
            nxt = (k + 1) % 2
            copies[nxt] = pltpu.async_copy(
                loss_hbm.at[pl.ds(base + (k + 1) * _CROWS, _CROWS)],
                buf.at[nxt], sems[nxt])
        copies[cur].wait()

        @plsc.parallel_loop(0, _VPC, unroll=8, carry=carry)
        def carry(i, cy):
            rr = i >> 5
            cc = (i & 31) * _L
            return process_vec(buf[cur, rr, pl.ds(cc, _L)], cy)

    return carry


@functools.partial(
    pl.kernel,
    out_type=jax.ShapeDtypeStruct((_NC, _ROWS, _COLS), jnp.int32),
    mesh=_MESH,
    compiler_params=_SC_PARAMS,
    scratch_types=[
        pltpu.VMEM((2, _CROWS, _LCOL), jnp.float32),
        pltpu.VMEM((_ROWS, _COLS), jnp.int32),
        pltpu.VMEM((_ROWS,), jnp.int32),
        pltpu.VMEM_SHARED((_ROWS, _COLS), jnp.int32),
        pltpu.SemaphoreType.DMA,
        pltpu.SemaphoreType.DMA,
    ],
)
def _sc_hist1(loss_hbm, outc_hbm, buf, h1c, rowidx, shc, sem0, sem1):
    cid = lax.axis_index("c")
    sid = lax.axis_index("s")
    _zero_hist1(h1c)
    _fill_rowidx(rowidx)

    @pl.when(sid == 0)
    def _zero_shared():
        pltpu.sync_copy(h1c, shc)

    plsc.subcore_barrier()

    ones = jnp.ones((_L,), jnp.int32)
    base = (sid * _NC + cid) * _WROWS

    def process(v, cy):
        p = lax.bitcast_convert_type(v, jnp.int32)
        b = jnp.right_shift(p, 19)
        r = jnp.right_shift(b, 7)
        c = jnp.bitwise_and(b, 127)
        plsc.addupdate_scatter(h1c, [r, c], ones)
        return cy

    _stream_chunks(loss_hbm, buf, (sem0, sem1), base, process,
                   jnp.int32(0))

    pltpu.sync_copy(h1c, shc.at[rowidx], add=True)
    plsc.subcore_barrier()

    @pl.when(sid == 0)
    def _write_out():
        pltpu.sync_copy(shc, outc_hbm.at[cid])


@functools.partial(
    pl.kernel,
    out_type=[
        jax.ShapeDtypeStruct((_NC, _ROWS, _COLS), jnp.int32),
        jax.ShapeDtypeStruct((_NC, _ROWS, _COLS), jnp.float32),
        jax.ShapeDtypeStruct((_L,), jnp.int32),
        jax.ShapeDtypeStruct((_NC, _L), jnp.float32),
    ],
    mesh=_MESH,
    compiler_params=_SC_PARAMS,
    scratch_types=[
        pltpu.VMEM((2, _CROWS, _LCOL), jnp.float32),
        pltpu.VMEM((_ROWS, _COLS), jnp.int32),
        pltpu.VMEM((_ROWS, _COLS), jnp.float32),
        pltpu.VMEM((_ROWS, _COLS), jnp.int32),
        pltpu.VMEM((_ROWS,), jnp.int32),
        pltpu.VMEM((_L,), jnp.int32),
        pltpu.VMEM((_L,), jnp.float32),
        pltpu.VMEM((_NS, _L), jnp.float32),
        pltpu.VMEM_SHARED((_ROWS, _COLS), jnp.int32),
        pltpu.VMEM_SHARED((_ROWS, _COLS), jnp.float32),
        pltpu.VMEM_SHARED((_NS, _L), jnp.float32),
        pltpu.SemaphoreType.DMA,
        pltpu.SemaphoreType.DMA,
    ],
)
def _sc_hist2(loss_hbm, h1c_hbm,
              outc_hbm, outs_hbm, auxi_hbm, auxf_hbm,
              buf, h2c, h2s, tmpc, rowidx, auxi_v, auxf_v, acc16,
              shc, shs, sh_acc, sem0, sem1):
    cid = lax.axis_index("c")
    sid = lax.axis_index("s")
    _fill_rowidx(rowidx)

    pltpu.sync_copy(h1c_hbm.at[0], h2c)
    pltpu.sync_copy(h1c_hbm.at[1], tmpc)
    _merge_into([h2c], [tmpc])

    b1, pc1, _, tot_c, _ = _scan_hist(h2c, None, jnp.int32(_N_TOT - _N_MIN))
    c_above = tot_c - pc1
    j_need = _N_MIN - c_above

    @pl.when(jnp.logical_and(sid == 0, cid == 0))
    def _write_aux():
        auxi_v[pl.ds(0, _L)] = (jnp.zeros((_L,), jnp.int32)
                                + jnp.where(lax.iota(jnp.int32, _L) == 0,
                                            b1, j_need))
        pltpu.sync_copy(auxi_v, auxi_hbm)

    _zero_hist2(h2c, h2s)

    @pl.when(sid == 0)
    def _zero_shared():
        pltpu.sync_copy(h2c, shc)
        pltpu.sync_copy(h2s, shs)

    plsc.subcore_barrier()

    ones = jnp.ones((_L,), jnp.int32)
    zf = jnp.zeros((_L,), jnp.float32)
    base = (sid * _NC + cid) * _WROWS

    def process(v, acc):
        p = lax.bitcast_convert_type(v, jnp.int32)
        b = jnp.right_shift(p, 19)
        meq = b == b1
        acc = acc + jnp.where(b > b1, v, zf)
        sub = jnp.bitwise_and(jnp.right_shift(p, 7), 4095)
        r = jnp.right_shift(sub, 7)
        c = jnp.bitwise_and(sub, 127)
        plsc.addupdate_scatter(h2c, [r, c], ones, mask=meq)
        plsc.addupdate_scatter(h2s, [r, c], v, mask=meq)
        return acc

    acc = _stream_chunks(loss_hbm, buf, (sem0, sem1), base, process, zf)

    auxf_v[pl.ds(0, _L)] = acc
    pltpu.sync_copy(auxf_v, sh_acc.at[sid])
    pltpu.sync_copy(h2c, shc.at[rowidx], add=True)
    pltpu.sync_copy(h2s, shs.at[rowidx], add=True)
    plsc.subcore_barrier()

    @pl.when(sid == 0)
    def _write_out():
        pltpu.sync_copy(shc, outc_hbm.at[cid])
        pltpu.sync_copy(shs, outs_hbm.at[cid])
        pltpu.sync_copy(sh_acc, acc16)
        total = acc16[0, pl.ds(0, _L)]
        for s in range(1, _NS):
            total = total + acc16[s, pl.ds(0, _L)]
        auxf_v[pl.ds(0, _L)] = total
        pltpu.sync_copy(auxf_v, auxf_hbm.at[cid])


@functools.partial(
    pl.kernel,
    out_type=jax.ShapeDtypeStruct((_L,), jnp.float32),
    mesh=_MESH,
    compiler_params=_SC_PARAMS,
    scratch_types=[
        pltpu.VMEM((_ROWS, _COLS), jnp.int32),
        pltpu.VMEM((_ROWS, _COLS), jnp.float32),
        pltpu.VMEM((_ROWS, _COLS), jnp.int32),
        pltpu.VMEM((_ROWS, _COLS), jnp.float32),
        pltpu.VMEM((1, _L), jnp.float32),
        pltpu.VMEM((_L,), jnp.int32),
        pltpu.VMEM((_NC, _L), jnp.float32),
        pltpu.VMEM((_L,), jnp.float32),
    ],
)
def _sc_final(h2c_hbm, h2s_hbm, auxi_hbm, auxf_hbm, stats_hbm, out_hbm,
              mc, ms, tmpc, tmps, stats_v, auxi_v, auxf_v, out_v):
    cid = lax.axis_index("c")
    sid = lax.axis_index("s")

    @pl.when(jnp.logical_and(sid == 0, cid == 0))
    def _go():
        pltpu.sync_copy(h2c_hbm.at[0], mc)
        pltpu.sync_copy(h2s_hbm.at[0], ms)
        pltpu.sync_copy(h2c_hbm.at[1], tmpc)
        pltpu.sync_copy(h2s_hbm.at[1], tmps)
        _merge_into([mc, ms], [tmpc, tmps])
        pltpu.sync_copy(auxi_hbm, auxi_v)
        pltpu.sync_copy(auxf_hbm, auxf_v)
        pltpu.sync_copy(stats_hbm, stats_v)

        auxi = auxi_v[pl.ds(0, _L)]
        b1 = _sval(auxi, 0)
        j_need = _sval(auxi, 1)
        s_above_v = auxf_v[0, pl.ds(0, _L)] + auxf_v[1, pl.ds(0, _L)]
        s_above = jnp.sum(s_above_v)

        _, _, _, tot_c2, tot_s2 = _scan_hist(mc, ms, jnp.int32(0x7FFFFFF0))
        c2, pc2, ps2, _, _ = _scan_hist(mc, ms, tot_c2 - j_need)
        a2 = tot_c2 - pc2
        s2 = tot_s2 - ps2
        rem = j_need - a2

        rep_bits = (jnp.zeros((_L,), jnp.int32)
                    + (jnp.left_shift(b1, 19) | jnp.left_shift(c2, 7)))
        rep = _sval(lax.bitcast_convert_type(rep_bits, jnp.float32), 0)
        sum_topk = s_above + s2 + rem.astype(jnp.float32) * rep
        mean_topk = sum_topk * jnp.float32(1.0 / _N_MIN)

        stats = stats_v[0, pl.ds(0, _L)]
        cnt_gt = _sval(stats, 1)
        mean_gt = _sval(stats, 2)

        final = jnp.where(cnt_gt > jnp.float32(_N_MIN), mean_gt, mean_topk)
        out_v[pl.ds(0, _L)] = jnp.zeros((_L,), jnp.float32) + final
        pltpu.sync_copy(out_v, out_hbm)


def kernel(logits, labels):
    loss, stats = _tc_loss(logits, labels)
    h1c = _sc_hist1(loss)
    h2c, h2s, auxi, auxf = _sc_hist2(loss, h1c)
    out = _sc_final(h2c, h2s, auxi, auxf, stats)
    return out[0]
